# baseline (speedup 1.0000x reference)
import functools

import jax
import jax.numpy as jnp
from jax import lax
from jax.experimental import pallas as pl
from jax.experimental.pallas import tpu as pltpu
from jax.experimental.pallas import tpu_sc as plsc

N = 10000
E = 320000
BM = 400
BMV = 400
NCORE = 2
NSUB = 16
LANES = 16
NW = NCORE * NSUB
EC = E // NW
NPAD = 10240
CHUNK = NPAD // NSUB


def _cast_body(b_ref, w_ref, wb_ref, h_ref):
    w = w_ref[...]
    wb_ref[...] = w.astype(jnp.bfloat16)
    acc = jnp.sum(w, axis=1, keepdims=True)
    h_ref[...] = jnp.maximum(acc + b_ref[...], 0.0)


def _cast_rowsum_relu(w, b2):
    wb, h = pl.pallas_call(
        _cast_body,
        grid=(N // BM,),
        in_specs=[
            pl.BlockSpec((BM, 1), lambda i: (i, 0)),
            pl.BlockSpec((BM, N), lambda i: (i, 0)),
        ],
        out_specs=[
            pl.BlockSpec((BM, N), lambda i: (i, 0)),
            pl.BlockSpec((BM, 1), lambda i: (i, 0)),
        ],
        out_shape=[
            jax.ShapeDtypeStruct((N, N), jnp.bfloat16),
            jax.ShapeDtypeStruct((N, 1), jnp.float32),
        ],
    )(b2, w)
    return wb, h


def _mv_body(x_ref, b_ref, w_ref, h_ref, s_ref):
    i = pl.program_id(0)
    x = x_ref[0:1, :] + x_ref[1:2, :]
    w = w_ref[...].astype(jnp.float32)
    acc = jnp.sum(w * x, axis=1, keepdims=True)
    h = jnp.maximum(acc + b_ref[...], 0.0)
    h_ref[...] = h
    part = jnp.sum(h).reshape(1, 1)

    @pl.when(i == 0)
    def _():
        s_ref[...] = part

    @pl.when(i != 0)
    def _():
        s_ref[...] += part


def _matvec_relu(wb, x2, b2):
    h, s = pl.pallas_call(
        _mv_body,
        grid=(N // BMV,),
        in_specs=[
            pl.BlockSpec((2, N), lambda i: (0, 0)),
            pl.BlockSpec((BMV, 1), lambda i: (i, 0)),
            pl.BlockSpec((BMV, N), lambda i: (i, 0)),
        ],
        out_specs=[
            pl.BlockSpec((BMV, 1), lambda i: (i, 0)),
            pl.BlockSpec((1, 1), lambda i: (0, 0)),
        ],
        out_shape=[
            jax.ShapeDtypeStruct((N, 1), jnp.float32),
            jax.ShapeDtypeStruct((1, 1), jnp.float32),
        ],
    )(x2, b2, wb)
    return h, s[0, 0]


def _segsum_body(src_hbm, dst_hbm, h_hbm, out_hbm,
                 src_v, dst_v, h_v, acc_v, tmp_v, sum_v, shared):
    cid = lax.axis_index("c")
    sid = lax.axis_index("s")
    wid = cid * NSUB + sid

    pltpu.sync_copy(src_hbm.at[pl.ds(wid * EC, EC)], src_v)
    pltpu.sync_copy(dst_hbm.at[pl.ds(wid * EC, EC)], dst_v)
    pltpu.sync_copy(h_hbm, h_v)

    zeros16 = jnp.zeros((LANES,), jnp.float32)

    def zbody(i, c):
        acc_v[pl.ds(i * LANES, LANES)] = zeros16
        return c

    lax.fori_loop(0, NPAD // LANES, zbody, 0)

    def ebody(e, c):
        s = src_v[pl.ds(e * LANES, LANES)]
        d = dst_v[pl.ds(e * LANES, LANES)]
        vals = plsc.load_gather(h_v, [s])
        plsc.addupdate_scatter(acc_v, [d], vals)
        return c

    lax.fori_loop(0, EC // LANES, ebody, 0)

    pltpu.sync_copy(acc_v, shared.at[sid])
    plsc.subcore_barrier()

    pltpu.sync_copy(shared.at[0, pl.ds(sid * CHUNK, CHUNK)], sum_v)

    def sbody(t, c):
        pltpu.sync_copy(shared.at[t, pl.ds(sid * CHUNK, CHUNK)], tmp_v)

        def abody(j, c2):
            sum_v[pl.ds(j * LANES, LANES)] += tmp_v[pl.ds(j * LANES, LANES)]
            return c2

        lax.fori_loop(0, CHUNK // LANES, abody, 0)
        return c

    lax.fori_loop(1, NSUB, sbody, 0)

    pltpu.sync_copy(sum_v, out_hbm.at[cid, pl.ds(sid * CHUNK, CHUNK)])


@functools.cache
def _get_segsum():
    return pl.kernel(
        _segsum_body,
        out_type=jax.ShapeDtypeStruct((NCORE, NPAD), jnp.float32),
        mesh=plsc.VectorSubcoreMesh(core_axis_name="c", subcore_axis_name="s",
                                    num_cores=NCORE, num_subcores=NSUB),
        compiler_params=pltpu.CompilerParams(use_tc_tiling_on_sc=False,
                                             needs_layout_passes=False),
        scratch_types=[
            pltpu.VMEM((EC,), jnp.int32),
            pltpu.VMEM((EC,), jnp.int32),
            pltpu.VMEM((N,), jnp.float32),
            pltpu.VMEM((NPAD,), jnp.float32),
            pltpu.VMEM((CHUNK,), jnp.float32),
            pltpu.VMEM((CHUNK,), jnp.float32),
            pltpu.VMEM_SHARED((NSUB, NPAD), jnp.float32),
        ],
    )


def _segment_sum(src, dst, h):
    return _get_segsum()(src, dst, h)[:, :N]


def kernel(edge_index, weight, bias):
    src = edge_index[0].astype(jnp.int32)
    dst = edge_index[1].astype(jnp.int32)
    b2 = bias.reshape(N, 1)

    dummy = _get_segsum()(src, dst, jnp.zeros((N,), jnp.float32))[:, :N]
    wb, h2 = _cast_rowsum_relu(weight, b2)
    agg2 = _segment_sum(src, dst, h2.reshape(N)) + dummy
    h1, _ = _matvec_relu(wb, agg2, b2)
    agg1 = _segment_sum(src, dst, h1.reshape(N))
    _, total = _matvec_relu(wb, agg1, b2)
    return total


# speedup vs baseline: 1.0695x; 1.0695x over previous
import functools

import jax
import jax.numpy as jnp
from jax import lax
from jax.experimental import pallas as pl
from jax.experimental.pallas import tpu as pltpu
from jax.experimental.pallas import tpu_sc as plsc

N = 10000
E = 320000
BM = 400
BMV = 400
NCORE = 2
NSUB = 16
LANES = 16
NW = NCORE * NSUB
EC = E // NW
NPAD = 10240
CHUNK = NPAD // NSUB


def _cast_body(b_ref, w_ref, wb_ref, h_ref):
    w = w_ref[...]
    wb_ref[...] = w.astype(jnp.bfloat16)
    acc = jnp.sum(w, axis=1, keepdims=True)
    h_ref[...] = jnp.maximum(acc + b_ref[...], 0.0)


def _cast_rowsum_relu(w, b2):
    wb, h = pl.pallas_call(
        _cast_body,
        grid=(N // BM,),
        in_specs=[
            pl.BlockSpec((BM, 1), lambda i: (i, 0)),
            pl.BlockSpec((BM, N), lambda i: (i, 0)),
        ],
        out_specs=[
            pl.BlockSpec((BM, N), lambda i: (i, 0)),
            pl.BlockSpec((BM, 1), lambda i: (i, 0)),
        ],
        out_shape=[
            jax.ShapeDtypeStruct((N, N), jnp.bfloat16),
            jax.ShapeDtypeStruct((N, 1), jnp.float32),
        ],
    )(b2, w)
    return wb, h


def _mv_body(x_ref, b_ref, w_ref, h_ref, s_ref):
    i = pl.program_id(0)
    x = x_ref[0:1, :N] + x_ref[1:2, :N]
    w = w_ref[...].astype(jnp.float32)
    acc = jnp.sum(w * x, axis=1, keepdims=True)
    h = jnp.maximum(acc + b_ref[...], 0.0)
    h_ref[...] = h
    part = jnp.sum(h).reshape(1, 1)

    @pl.when(i == 0)
    def _():
        s_ref[...] = part

    @pl.when(i != 0)
    def _():
        s_ref[...] += part


def _matvec_relu(wb, x2, b2):
    h, s = pl.pallas_call(
        _mv_body,
        grid=(N // BMV,),
        in_specs=[
            pl.BlockSpec((2, NPAD), lambda i: (0, 0)),
            pl.BlockSpec((BMV, 1), lambda i: (i, 0)),
            pl.BlockSpec((BMV, N), lambda i: (i, 0)),
        ],
        out_specs=[
            pl.BlockSpec((BMV, 1), lambda i: (i, 0)),
            pl.BlockSpec((1, 1), lambda i: (0, 0)),
        ],
        out_shape=[
            jax.ShapeDtypeStruct((N, 1), jnp.float32),
            jax.ShapeDtypeStruct((1, 1), jnp.float32),
        ],
    )(x2, b2, wb)
    return h, s[0, 0]


def _segsum_body(src_hbm, dst_hbm, h_hbm, out_hbm,
                 src_v, dst_v, h_v, acc_v, tmp_v, sum_v, shared):
    cid = lax.axis_index("c")
    sid = lax.axis_index("s")
    wid = cid * NSUB + sid

    pltpu.sync_copy(src_hbm.at[pl.ds(wid * EC, EC)], src_v)
    pltpu.sync_copy(dst_hbm.at[pl.ds(wid * EC, EC)], dst_v)
    pltpu.sync_copy(h_hbm, h_v)

    zeros16 = jnp.zeros((LANES,), jnp.float32)

    def zbody(i, c):
        acc_v[pl.ds(i * LANES, LANES)] = zeros16
        return c

    lax.fori_loop(0, NPAD // LANES, zbody, 0)

    def ebody(e, c):
        s = src_v[pl.ds(e * LANES, LANES)]
        d = dst_v[pl.ds(e * LANES, LANES)]
        vals = plsc.load_gather(h_v, [s])
        plsc.addupdate_scatter(acc_v, [d], vals)
        return c

    lax.fori_loop(0, EC // LANES, ebody, 0)

    pltpu.sync_copy(acc_v, shared.at[sid])
    plsc.subcore_barrier()

    pltpu.sync_copy(shared.at[:, pl.ds(sid * CHUNK, CHUNK)], tmp_v)

    def sbody(j, c):
        def rbody(t, v):
            return v + tmp_v[t, pl.ds(j * LANES, LANES)]

        sum_v[pl.ds(j * LANES, LANES)] = lax.fori_loop(
            1, NSUB, rbody, tmp_v[0, pl.ds(j * LANES, LANES)])
        return c

    lax.fori_loop(0, CHUNK // LANES, sbody, 0)

    pltpu.sync_copy(sum_v, out_hbm.at[cid, pl.ds(sid * CHUNK, CHUNK)])


@functools.cache
def _get_segsum():
    return pl.kernel(
        _segsum_body,
        out_type=jax.ShapeDtypeStruct((NCORE, NPAD), jnp.float32),
        mesh=plsc.VectorSubcoreMesh(core_axis_name="c", subcore_axis_name="s",
                                    num_cores=NCORE, num_subcores=NSUB),
        compiler_params=pltpu.CompilerParams(use_tc_tiling_on_sc=False,
                                             needs_layout_passes=False),
        scratch_types=[
            pltpu.VMEM((EC,), jnp.int32),
            pltpu.VMEM((EC,), jnp.int32),
            pltpu.VMEM((N,), jnp.float32),
            pltpu.VMEM((NPAD,), jnp.float32),
            pltpu.VMEM((NSUB, CHUNK), jnp.float32),
            pltpu.VMEM((CHUNK,), jnp.float32),
            pltpu.VMEM_SHARED((NSUB, NPAD), jnp.float32),
        ],
    )


def _segment_sum(src, dst, h):
    return _get_segsum()(src, dst, h)


def kernel(edge_index, weight, bias):
    src = edge_index[0].astype(jnp.int32)
    dst = edge_index[1].astype(jnp.int32)
    b2 = bias.reshape(N, 1)

    wb, h2 = _cast_rowsum_relu(weight, b2)
    agg2 = _segment_sum(src, dst, h2.reshape(N))
    h1, _ = _matvec_relu(wb, agg2, b2)
    agg1 = _segment_sum(src, dst, h1.reshape(N))
    _, total = _matvec_relu(wb, agg1, b2)
    return total


# speedup vs baseline: 1.0885x; 1.0177x over previous
import functools

import jax
import jax.numpy as jnp
from jax import lax
from jax.experimental import pallas as pl
from jax.experimental.pallas import tpu as pltpu
from jax.experimental.pallas import tpu_sc as plsc

N = 10000
E = 320000
BM = 400
BMV = 400
NCORE = 2
NSUB = 16
LANES = 16
NW = NCORE * NSUB
EC = E // NW
NPAD = 10240
CHUNK = NPAD // NSUB


def _cast_body(b_ref, w_ref, wb_ref, h_ref):
    w = w_ref[...]
    wb_ref[...] = w.astype(jnp.bfloat16)
    acc = jnp.sum(w, axis=1, keepdims=True)
    h_ref[...] = jnp.maximum(acc + b_ref[...], 0.0)


def _cast_rowsum_relu(w, b2):
    wb, h = pl.pallas_call(
        _cast_body,
        grid=(N // BM,),
        in_specs=[
            pl.BlockSpec((BM, 1), lambda i: (i, 0)),
            pl.BlockSpec((BM, N), lambda i: (i, 0)),
        ],
        out_specs=[
            pl.BlockSpec((BM, N), lambda i: (i, 0)),
            pl.BlockSpec((BM, 1), lambda i: (i, 0)),
        ],
        out_shape=[
            jax.ShapeDtypeStruct((N, N), jnp.bfloat16),
            jax.ShapeDtypeStruct((N, 1), jnp.float32),
        ],
    )(b2, w)
    return wb, h


def _mv_body(x_ref, b_ref, w_ref, h_ref, s_ref):
    i = pl.program_id(0)
    x = x_ref[0:1, :N] + x_ref[1:2, :N]
    w = w_ref[...].astype(jnp.float32)
    acc = jnp.sum(w * x, axis=1, keepdims=True)
    h = jnp.maximum(acc + b_ref[...], 0.0)
    h_ref[...] = h
    part = jnp.sum(h).reshape(1, 1)

    @pl.when(i == 0)
    def _():
        s_ref[...] = part

    @pl.when(i != 0)
    def _():
        s_ref[...] += part


def _matvec_relu(wb, x2, b2):
    h, s = pl.pallas_call(
        _mv_body,
        grid=(N // BMV,),
        in_specs=[
            pl.BlockSpec((2, NPAD), lambda i: (0, 0)),
            pl.BlockSpec((BMV, 1), lambda i: (i, 0)),
            pl.BlockSpec((BMV, N), lambda i: (i, 0)),
        ],
        out_specs=[
            pl.BlockSpec((BMV, 1), lambda i: (i, 0)),
            pl.BlockSpec((1, 1), lambda i: (0, 0)),
        ],
        out_shape=[
            jax.ShapeDtypeStruct((N, 1), jnp.float32),
            jax.ShapeDtypeStruct((1, 1), jnp.float32),
        ],
    )(x2, b2, wb)
    return h, s[0, 0]


def _segsum_body(src_hbm, dst_hbm, h_hbm, out_hbm,
                 src_v, dst_v, h_v, acc_v, tmp_v, sum_v, shared,
                 sem1, sem2, sem3):
    cid = lax.axis_index("c")
    sid = lax.axis_index("s")
    wid = cid * NSUB + sid

    c1 = pltpu.async_copy(src_hbm.at[pl.ds(wid * EC, EC)], src_v, sem1)
    c2 = pltpu.async_copy(dst_hbm.at[pl.ds(wid * EC, EC)], dst_v, sem2)
    c3 = pltpu.async_copy(h_hbm, h_v, sem3)

    zeros16 = jnp.zeros((LANES,), jnp.float32)

    def zbody(i, c):
        acc_v[pl.ds(i * LANES, LANES)] = zeros16
        return c

    lax.fori_loop(0, NPAD // LANES, zbody, 0)
    c1.wait()
    c2.wait()
    c3.wait()

    def ebody(e, c):
        s = src_v[pl.ds(e * LANES, LANES)]
        d = dst_v[pl.ds(e * LANES, LANES)]
        vals = plsc.load_gather(h_v, [s])
        plsc.addupdate_scatter(acc_v, [d], vals)
        return c

    lax.fori_loop(0, EC // LANES, ebody, 0)

    pltpu.sync_copy(acc_v, shared.at[sid])
    plsc.subcore_barrier()

    pltpu.sync_copy(shared.at[:, pl.ds(sid * CHUNK, CHUNK)], tmp_v)

    def sbody(j, c):
        def rbody(t, v):
            return v + tmp_v[t, pl.ds(j * LANES, LANES)]

        sum_v[pl.ds(j * LANES, LANES)] = lax.fori_loop(
            1, NSUB, rbody, tmp_v[0, pl.ds(j * LANES, LANES)])
        return c

    lax.fori_loop(0, CHUNK // LANES, sbody, 0)

    pltpu.sync_copy(sum_v, out_hbm.at[cid, pl.ds(sid * CHUNK, CHUNK)])


@functools.cache
def _get_segsum():
    return pl.kernel(
        _segsum_body,
        out_type=jax.ShapeDtypeStruct((NCORE, NPAD), jnp.float32),
        mesh=plsc.VectorSubcoreMesh(core_axis_name="c", subcore_axis_name="s",
                                    num_cores=NCORE, num_subcores=NSUB),
        compiler_params=pltpu.CompilerParams(use_tc_tiling_on_sc=False,
                                             needs_layout_passes=False),
        scratch_types=[
            pltpu.VMEM((EC,), jnp.int32),
            pltpu.VMEM((EC,), jnp.int32),
            pltpu.VMEM((N,), jnp.float32),
            pltpu.VMEM((NPAD,), jnp.float32),
            pltpu.VMEM((NSUB, CHUNK), jnp.float32),
            pltpu.VMEM((CHUNK,), jnp.float32),
            pltpu.VMEM_SHARED((NSUB, NPAD), jnp.float32),
            pltpu.SemaphoreType.DMA,
            pltpu.SemaphoreType.DMA,
            pltpu.SemaphoreType.DMA,
        ],
    )


def _segment_sum(src, dst, h):
    return _get_segsum()(src, dst, h)


def kernel(edge_index, weight, bias):
    src = edge_index[0].astype(jnp.int32)
    dst = edge_index[1].astype(jnp.int32)
    b2 = bias.reshape(N, 1)

    wb, h2 = _cast_rowsum_relu(weight, b2)
    agg2 = _segment_sum(src, dst, h2.reshape(N))
    h1, _ = _matvec_relu(wb, agg2, b2)
    agg1 = _segment_sum(src, dst, h1.reshape(N))
    _, total = _matvec_relu(wb, agg1, b2)
    return total
